# baseline (speedup 1.0000x reference)
import functools
import math

import numpy as np
import jax
import jax.numpy as jnp
from jax import lax
from jax.experimental import pallas as pl
from jax.experimental.pallas import tpu as pltpu
from jax.experimental.pallas import tpu_sc as plsc

_VSZ = 1000000
_DSZ = 64
_B = 1024
_T = 200
_ROWS = _B * _T
_NW = 32
_PER_W = _ROWS // _NW
_CHUNK = 128
_NCHUNK = _PER_W // _CHUNK
_SCALE = math.sqrt(_DSZ)
_PEROWS = 320


def _build_pe_pairs() -> np.ndarray:
    log_timescale_increment = math.log(10000.0) / float(_DSZ)
    inv_timescales = np.exp(
        np.arange(0, _DSZ, 2, dtype=np.float32) * -log_timescale_increment)
    pe = np.zeros((_T, _DSZ), dtype=np.float32)
    position = np.arange(0, _T, dtype=np.float32)[:, None]
    pe[:, 0::2] = np.sin(position * inv_timescales)
    pe[:, 1::2] = np.cos(position * inv_timescales)
    big = pe[np.arange(_PEROWS) % _T]
    return big.reshape(_PEROWS // 2, 2 * _DSZ)


_PE_PAIRS = _build_pe_pairs()

_mesh = plsc.VectorSubcoreMesh(core_axis_name="c", subcore_axis_name="s")


_NBLK = _VSZ // 8
_CBLK = 32
_BASEBLK = _NBLK // _NW
_EXTRA = _NBLK - _BASEBLK * _NW


@functools.partial(
    pl.kernel,
    mesh=_mesh,
    compiler_params=pltpu.CompilerParams(use_tc_tiling_on_sc=True),
    out_type=jax.ShapeDtypeStruct((_VSZ // 2, 2 * _DSZ), jnp.float32),
    scratch_types=[
        pltpu.VMEM((_CBLK * 8, _DSZ), jnp.float32),
        pltpu.VMEM((_CBLK * 8, _DSZ), jnp.float32),
        pltpu.VMEM((_CBLK * 4, 2 * _DSZ), jnp.float32),
        pltpu.VMEM((_CBLK * 4, 2 * _DSZ), jnp.float32),
        pltpu.SemaphoreType.DMA,
        pltpu.SemaphoreType.DMA,
        pltpu.SemaphoreType.DMA,
        pltpu.SemaphoreType.DMA,
    ],
)
def _sc_compact(w_hbm, wc_hbm, in_a, in_b, dn_a, dn_b, sem_ra, sem_rb,
                sem_wa, sem_wb):
    wid = lax.axis_index("s") * 2 + lax.axis_index("c")
    nblk = _BASEBLK + 2 * jnp.where(wid < _EXTRA // 2, 1, 0)
    sblk = _BASEBLK * wid + 2 * jnp.minimum(wid, _EXTRA // 2)
    niter = (_BASEBLK + 2 + _CBLK - 1) // _CBLK + 1

    def start(i):
        return sblk + jnp.minimum(i * _CBLK, nblk - _CBLK)

    def rstart(i, buf, sem):
        r0 = pl.multiple_of(start(i) * 8, 16)
        pltpu.async_copy(w_hbm.at[pl.ds(r0, _CBLK * 8)], buf, sem)

    def rwait(buf, sem):
        pltpu.make_async_copy(w_hbm.at[pl.ds(0, _CBLK * 8)], buf, sem).wait()

    def wstart(i, buf, sem):
        p0 = pl.multiple_of(start(i) * 4, 8)
        pltpu.async_copy(buf, wc_hbm.at[pl.ds(p0, _CBLK * 4)], sem)

    def wwait(buf, sem):
        pltpu.make_async_copy(buf, wc_hbm.at[pl.ds(0, _CBLK * 4)], sem).wait()

    def compact(in_v, dn_v):
        def pair_body(j, carry):
            for par in range(2):
                for q in range(_DSZ // 16):
                    dn_v[j, pl.ds(par * _DSZ + q * 16, 16)] = (
                        in_v[2 * j + par, pl.ds(q * 16, 16)])
            return carry

        lax.fori_loop(0, _CBLK * 4, pair_body, 0)

    rstart(0, in_a, sem_ra)

    def body(k, carry):
        i0 = 2 * k
        i1 = 2 * k + 1
        rstart(i1, in_b, sem_rb)
        rwait(in_a, sem_ra)

        @pl.when(k > 0)
        def _():
            wwait(dn_a, sem_wa)

        compact(in_a, dn_a)
        wstart(i0, dn_a, sem_wa)

        @pl.when(k < niter // 2 - 1)
        def _():
            rstart(i0 + 2, in_a, sem_ra)

        rwait(in_b, sem_rb)

        @pl.when(k > 0)
        def _():
            wwait(dn_b, sem_wb)

        compact(in_b, dn_b)
        wstart(i1, dn_b, sem_wb)
        return carry

    lax.fori_loop(0, niter // 2, body, 0)
    wwait(dn_a, sem_wa)
    wwait(dn_b, sem_wb)


@functools.partial(
    pl.kernel,
    mesh=_mesh,
    compiler_params=pltpu.CompilerParams(use_tc_tiling_on_sc=True),
    out_type=jax.ShapeDtypeStruct((_ROWS, _DSZ), jnp.float32),
    scratch_types=[
        pltpu.VMEM((_NCHUNK, 128), jnp.int32),
        pltpu.VMEM((_NCHUNK, 128), jnp.int32),
        pltpu.VMEM((_CHUNK, 128), jnp.float32),
        pltpu.VMEM((_CHUNK, 128), jnp.float32),
        pltpu.VMEM((_CHUNK, _DSZ), jnp.float32),
        pltpu.VMEM((_CHUNK, _DSZ), jnp.float32),
        pltpu.VMEM((_PEROWS // 2, 2 * _DSZ), jnp.float32),
        pltpu.SemaphoreType.DMA,
        pltpu.SemaphoreType.DMA,
        pltpu.SemaphoreType.DMA,
        pltpu.SemaphoreType.DMA,
    ],
)
def _sc_embed(w_hbm, idx_hbm, idxp_hbm, pe_hbm, out_hbm, idx_v, idxp_v,
              rows_a, rows_b, out_a, out_b, pe_v, sem_ga, sem_gb, sem_wa,
              sem_wb):
    wid = lax.axis_index("s") * 2 + lax.axis_index("c")
    pltpu.sync_copy(pe_hbm, pe_v)
    pltpu.sync_copy(idx_hbm.at[wid], idx_v)
    pltpu.sync_copy(idxp_hbm.at[wid], idxp_v)

    def gstart(c, slot, buf, sem):
        pltpu.async_copy(w_hbm.at[idxp_v.at[c]], buf, sem)

    def gwait(buf, sem):
        pltpu.make_async_copy(w_hbm.at[idxp_v.at[0]], buf, sem).wait()

    def wstart(c, buf, sem):
        base = pl.multiple_of(wid * _PER_W + c * _CHUNK, _CHUNK)
        pltpu.async_copy(buf, out_hbm.at[pl.ds(base, _CHUNK)], sem)

    def wwait(buf, sem):
        pltpu.make_async_copy(buf, out_hbm.at[pl.ds(0, _CHUNK)], sem).wait()

    def compute(c, rows_v, out_v):
        base = wid * _PER_W + c * _CHUNK
        off0 = base % _T

        def grp_body(g, carry):
            iv = idx_v[c, pl.ds(g * 16, 16)]
            sv = jnp.where(iv == 0, 0.0, _SCALE).astype(jnp.float32)
            hv = (iv & 1) * _DSZ
            for rl in range(16):
                r = g * 16 + rl
                s_r = sv[rl]
                h_r = hv[rl]
                half = (rl % 2) * _DSZ
                pj = (off0 + r) // 2
                for q in range(_DSZ // 16):
                    out_v[r, pl.ds(q * 16, 16)] = (
                        rows_v[r, pl.ds(h_r + q * 16, 16)] * s_r
                        + pe_v[pj, pl.ds(half + q * 16, 16)]
                    )
            return carry

        lax.fori_loop(0, _CHUNK // 16, grp_body, 0)

    gstart(0, 0, rows_a, sem_ga)

    def body(k, carry):
        c0 = 2 * k
        c1 = 2 * k + 1
        gstart(c1, 1, rows_b, sem_gb)
        gwait(rows_a, sem_ga)

        @pl.when(k > 0)
        def _():
            wwait(out_a, sem_wa)

        compute(c0, rows_a, out_a)
        wstart(c0, out_a, sem_wa)

        @pl.when(k < _NCHUNK // 2 - 1)
        def _():
            gstart(c0 + 2, 0, rows_a, sem_ga)

        gwait(rows_b, sem_gb)

        @pl.when(k > 0)
        def _():
            wwait(out_b, sem_wb)

        compute(c1, rows_b, out_b)
        wstart(c1, out_b, sem_wb)
        return carry

    lax.fori_loop(0, _NCHUNK // 2, body, 0)
    wwait(out_a, sem_wa)
    wwait(out_b, sem_wb)


def kernel(x, W):
    B, T = x.shape
    assert (B, T) == (_B, _T) and W.shape == (_VSZ, _DSZ)
    xf = x.reshape(_NW, _NCHUNK, 128).astype(jnp.int32)
    xp = lax.shift_right_logical(xf, 1)
    w2 = _sc_compact(W.astype(jnp.float32))
    out = _sc_embed(w2, xf, xp, jnp.asarray(_PE_PAIRS))
    return out.reshape(B, T, _DSZ)


# speedup vs baseline: 1.2357x; 1.2357x over previous
import functools
import math

import numpy as np
import jax
import jax.numpy as jnp
from jax import lax
from jax.experimental import pallas as pl
from jax.experimental.pallas import tpu as pltpu
from jax.experimental.pallas import tpu_sc as plsc

_VSZ = 1000000
_DSZ = 64
_B = 1024
_T = 200
_ROWS = _B * _T
_NW = 32
_PER_W = _ROWS // _NW
_CHUNK = 128
_NCHUNK = _PER_W // _CHUNK
_SCALE = math.sqrt(_DSZ)
_PEROWS = 216


def _build_pe_pairs() -> np.ndarray:
    log_timescale_increment = math.log(10000.0) / float(_DSZ)
    inv_timescales = np.exp(
        np.arange(0, _DSZ, 2, dtype=np.float32) * -log_timescale_increment)
    pe = np.zeros((_T, _DSZ), dtype=np.float32)
    position = np.arange(0, _T, dtype=np.float32)[:, None]
    pe[:, 0::2] = np.sin(position * inv_timescales)
    pe[:, 1::2] = np.cos(position * inv_timescales)
    big = pe[np.arange(_PEROWS) % _T]
    return big.reshape(_PEROWS // 2, 2 * _DSZ)


_PE_PAIRS = _build_pe_pairs()

_mesh = plsc.VectorSubcoreMesh(core_axis_name="c", subcore_axis_name="s")


@functools.partial(
    pl.kernel,
    mesh=_mesh,
    compiler_params=pltpu.CompilerParams(use_tc_tiling_on_sc=True),
    out_type=jax.ShapeDtypeStruct((_ROWS, _DSZ), jnp.float32),
    scratch_types=[
        pltpu.VMEM((_NCHUNK, 128), jnp.int32),
        pltpu.VMEM((_NCHUNK, 128), jnp.int32),
        pltpu.VMEM((_CHUNK, 128), jnp.float32),
        pltpu.VMEM((_CHUNK, 128), jnp.float32),
        pltpu.VMEM((_CHUNK, 128), jnp.float32),
        pltpu.VMEM((_CHUNK, 128), jnp.float32),
        pltpu.VMEM((_CHUNK, _DSZ), jnp.float32),
        pltpu.VMEM((_CHUNK, _DSZ), jnp.float32),
        pltpu.VMEM((_PEROWS // 2, 2 * _DSZ), jnp.float32),
        pltpu.SemaphoreType.DMA,
        pltpu.SemaphoreType.DMA,
        pltpu.SemaphoreType.DMA,
        pltpu.SemaphoreType.DMA,
        pltpu.SemaphoreType.DMA,
        pltpu.SemaphoreType.DMA,
    ],
)
def _sc_embed(w_hbm, idx_hbm, idxp_hbm, pe_hbm, out_hbm, idx_v, idxp_v,
              rows_a, rows_b, rows_c, rows_d, out_a, out_b, pe_v, sem_ga,
              sem_gb, sem_gc, sem_gd, sem_wa, sem_wb):
    wid = lax.axis_index("s") * 2 + lax.axis_index("c")
    pltpu.sync_copy(pe_hbm, pe_v)
    pltpu.sync_copy(idx_hbm.at[wid], idx_v)
    pltpu.sync_copy(idxp_hbm.at[wid], idxp_v)

    def gstart(c, slot, buf, sem):
        pltpu.async_copy(w_hbm.at[idxp_v.at[c]], buf, sem)

    def gwait(buf, sem):
        pltpu.make_async_copy(w_hbm.at[idxp_v.at[0]], buf, sem).wait()

    def wstart(c, buf, sem):
        base = pl.multiple_of(wid * _PER_W + c * _CHUNK, _CHUNK)
        pltpu.async_copy(buf, out_hbm.at[pl.ds(base, _CHUNK)], sem)

    def wwait(buf, sem):
        pltpu.make_async_copy(buf, out_hbm.at[pl.ds(0, _CHUNK)], sem).wait()

    def compute(c, rows_v, out_v):
        base = wid * _PER_W + c * _CHUNK

        def grp_body(g, carry):
            iv = idx_v[c, pl.ds(g * 16, 16)]
            sv = jnp.where(iv == 0, 0.0, _SCALE).astype(jnp.float32)
            hv = (iv & 1) * _DSZ
            t0 = (base + g * 16) % _T
            for rl in range(16):
                r = g * 16 + rl
                s_r = sv[rl]
                h_r = hv[rl]
                half = (rl % 2) * _DSZ
                pj = (t0 + rl) // 2
                for q in range(_DSZ // 16):
                    out_v[r, pl.ds(q * 16, 16)] = (
                        rows_v[r, pl.ds(h_r + q * 16, 16)] * s_r
                        + pe_v[pj, pl.ds(half + q * 16, 16)]
                    )
            return carry

        lax.fori_loop(0, _CHUNK // 16, grp_body, 0)

    rbufs = ((rows_a, sem_ga), (rows_b, sem_gb), (rows_c, sem_gc),
             (rows_d, sem_gd))
    obufs = ((out_a, sem_wa), (out_b, sem_wb))

    for j in range(3):
        gstart(j, 0, rbufs[j][0], rbufs[j][1])

    def body(k, carry):
        gstart(4 * k + 3, 0, rbufs[3][0], rbufs[3][1])
        for j in range(4):
            c = 4 * k + j
            rb, rs = rbufs[j]
            ob, os_ = obufs[j % 2]
            gwait(rb, rs)

            @pl.when(c >= 2)
            def _():
                wwait(ob, os_)

            compute(c, rb, ob)
            wstart(c, ob, os_)

            @pl.when(c + 4 < _NCHUNK)
            def _():
                gstart(c + 4, 0, rb, rs)

        return carry

    lax.fori_loop(0, _NCHUNK // 4, body, 0)
    for c in (_NCHUNK - 2, _NCHUNK - 1):
        rb, rs = rbufs[c % 4]
        ob, os_ = obufs[c % 2]
        gwait(rb, rs)
        wwait(ob, os_)
        compute(c, rb, ob)
        wstart(c, ob, os_)
    wwait(out_a, sem_wa)
    wwait(out_b, sem_wb)


def kernel(x, W):
    B, T = x.shape
    assert (B, T) == (_B, _T) and W.shape == (_VSZ, _DSZ)
    xf = x.reshape(_NW, _NCHUNK, 128).astype(jnp.int32)
    xp = lax.shift_right_logical(xf, 1)
    w2 = W.astype(jnp.float32).reshape(_VSZ // 2, 2 * _DSZ)
    out = _sc_embed(w2, xf, xp, jnp.asarray(_PE_PAIRS))
    return out.reshape(B, T, _DSZ)


# speedup vs baseline: 1.2407x; 1.0040x over previous
import functools
import math

import numpy as np
import jax
import jax.numpy as jnp
from jax import lax
from jax.experimental import pallas as pl
from jax.experimental.pallas import tpu as pltpu
from jax.experimental.pallas import tpu_sc as plsc

_VSZ = 1000000
_DSZ = 64
_B = 1024
_T = 200
_ROWS = _B * _T
_NW = 32
_PER_W = _ROWS // _NW
_CHUNK = 128
_NCHUNK = _PER_W // _CHUNK
_SCALE = math.sqrt(_DSZ)
_PEROWS = 320


def _build_pe_pairs() -> np.ndarray:
    log_timescale_increment = math.log(10000.0) / float(_DSZ)
    inv_timescales = np.exp(
        np.arange(0, _DSZ, 2, dtype=np.float32) * -log_timescale_increment)
    pe = np.zeros((_T, _DSZ), dtype=np.float32)
    position = np.arange(0, _T, dtype=np.float32)[:, None]
    pe[:, 0::2] = np.sin(position * inv_timescales)
    pe[:, 1::2] = np.cos(position * inv_timescales)
    big = pe[np.arange(_PEROWS) % _T]
    return big.reshape(_PEROWS // 2, 2 * _DSZ)


_PE_PAIRS = _build_pe_pairs()

_mesh = plsc.VectorSubcoreMesh(core_axis_name="c", subcore_axis_name="s")


@functools.partial(
    pl.kernel,
    mesh=_mesh,
    compiler_params=pltpu.CompilerParams(use_tc_tiling_on_sc=True),
    out_type=jax.ShapeDtypeStruct((_ROWS, _DSZ), jnp.float32),
    scratch_types=[
        pltpu.VMEM((_NCHUNK, 128), jnp.int32),
        pltpu.VMEM((_NCHUNK, 128), jnp.int32),
        pltpu.VMEM((_CHUNK, 128), jnp.float32),
        pltpu.VMEM((_CHUNK, 128), jnp.float32),
        pltpu.VMEM((_CHUNK, _DSZ), jnp.float32),
        pltpu.VMEM((_CHUNK, _DSZ), jnp.float32),
        pltpu.VMEM((_PEROWS // 2, 2 * _DSZ), jnp.float32),
        pltpu.SemaphoreType.DMA,
        pltpu.SemaphoreType.DMA,
        pltpu.SemaphoreType.DMA,
        pltpu.SemaphoreType.DMA,
    ],
)
def _sc_embed(w_hbm, idx_hbm, idxp_hbm, pe_hbm, out_hbm, idx_v, idxp_v,
              rows_a, rows_b, out_a, out_b, pe_v, sem_ga, sem_gb, sem_wa,
              sem_wb):
    wid = lax.axis_index("s") * 2 + lax.axis_index("c")
    pltpu.sync_copy(pe_hbm, pe_v)
    pltpu.sync_copy(idx_hbm.at[wid], idx_v)
    pltpu.sync_copy(idxp_hbm.at[wid], idxp_v)

    def gstart(c, slot, buf, sem):
        pltpu.async_copy(w_hbm.at[idxp_v.at[c]], buf, sem)

    def gwait(buf, sem):
        pltpu.make_async_copy(w_hbm.at[idxp_v.at[0]], buf, sem).wait()

    def wstart(c, buf, sem):
        base = pl.multiple_of(wid * _PER_W + c * _CHUNK, _CHUNK)
        pltpu.async_copy(buf, out_hbm.at[pl.ds(base, _CHUNK)], sem)

    def wwait(buf, sem):
        pltpu.make_async_copy(buf, out_hbm.at[pl.ds(0, _CHUNK)], sem).wait()

    def compute(c, rows_v, out_v):
        base = wid * _PER_W + c * _CHUNK
        off0 = base % _T

        def grp_body(g, carry):
            iv = idx_v[c, pl.ds(g * 16, 16)]
            sv = jnp.where(iv == 0, 0.0, _SCALE).astype(jnp.float32)
            hv = (iv & 1) * _DSZ
            for rl in range(16):
                r = g * 16 + rl
                s_r = sv[rl]
                h_r = hv[rl]
                half = (rl % 2) * _DSZ
                pj = (off0 + r) // 2
                for q in range(_DSZ // 16):
                    out_v[r, pl.ds(q * 16, 16)] = (
                        rows_v[r, pl.ds(h_r + q * 16, 16)] * s_r
                        + pe_v[pj, pl.ds(half + q * 16, 16)]
                    )
            return carry

        lax.fori_loop(0, _CHUNK // 16, grp_body, 0)

    gstart(0, 0, rows_a, sem_ga)

    def body(k, carry):
        c0 = 2 * k
        c1 = 2 * k + 1
        gstart(c1, 1, rows_b, sem_gb)
        gwait(rows_a, sem_ga)

        @pl.when(k > 0)
        def _():
            wwait(out_a, sem_wa)

        compute(c0, rows_a, out_a)
        wstart(c0, out_a, sem_wa)

        @pl.when(k < _NCHUNK // 2 - 1)
        def _():
            gstart(c0 + 2, 0, rows_a, sem_ga)

        gwait(rows_b, sem_gb)

        @pl.when(k > 0)
        def _():
            wwait(out_b, sem_wb)

        compute(c1, rows_b, out_b)
        wstart(c1, out_b, sem_wb)
        return carry

    lax.fori_loop(0, _NCHUNK // 2, body, 0)
    wwait(out_a, sem_wa)
    wwait(out_b, sem_wb)


def kernel(x, W):
    B, T = x.shape
    assert (B, T) == (_B, _T) and W.shape == (_VSZ, _DSZ)
    xf = x.reshape(_NW, _NCHUNK, 128).astype(jnp.int32)
    xp = lax.shift_right_logical(xf, 1)
    w2 = W.astype(jnp.float32).reshape(_VSZ // 2, 2 * _DSZ)
    out = _sc_embed(w2, xf, xp, jnp.asarray(_PE_PAIRS))
    return out.reshape(B, T, _DSZ)
